# baseline (speedup 1.0000x reference)
import jax
import jax.numpy as jnp
from jax.experimental import pallas as pl
from jax.experimental.pallas import tpu as pltpu

N_HALF = 25000
D = 256
BR = 5000
NB = N_HALF // BR


def _proj_kernel(ctx_ref, smp_ref, w_ref, b_ref, out_ref):
    i = pl.program_id(0)

    @pl.when(i < NB)
    def _():
        out_ref[...] = (
            jnp.dot(
                ctx_ref[...].astype(jnp.bfloat16),
                w_ref[...],
                preferred_element_type=jnp.float32,
            )
            + b_ref[...]
        )

    @pl.when(i >= NB)
    def _():
        out_ref[...] = (
            jnp.dot(
                smp_ref[...].astype(jnp.bfloat16),
                w_ref[...],
                preferred_element_type=jnp.float32,
            )
            + b_ref[...]
        )


def kernel(context, sample, W_proj, b_proj):
    b2d = b_proj.reshape(1, D)
    w_bf16 = W_proj.astype(jnp.bfloat16)
    out = pl.pallas_call(
        _proj_kernel,
        grid=(2 * NB,),
        in_specs=[
            pl.BlockSpec((BR, D), lambda i: (jnp.minimum(i, NB - 1), 0)),
            pl.BlockSpec((BR, D), lambda i: (jnp.maximum(i - NB, 0), 0)),
            pl.BlockSpec((D, D), lambda i: (0, 0)),
            pl.BlockSpec((1, D), lambda i: (0, 0)),
        ],
        out_specs=pl.BlockSpec((BR, D), lambda i: (i, 0)),
        out_shape=jax.ShapeDtypeStruct((2 * N_HALF, D), jnp.float32),
        compiler_params=pltpu.CompilerParams(
            dimension_semantics=("parallel",),
        ),
    )(context, sample, w_bf16, b2d)
    return out


# speedup vs baseline: 1.0591x; 1.0591x over previous
import jax
import jax.numpy as jnp
from jax.experimental import pallas as pl
from jax.experimental.pallas import tpu as pltpu

N_HALF = 25000
D = 256
BC = 1000
NCH = N_HALF // BC
NC = 2 * NCH
NBUF = 4


def _mm_kernel(ctx_hbm, smp_hbm, w_ref, b_ref, out_hbm, xbuf, obuf, in_sem, out_sem):
    def start_in(c, slot):
        @pl.when(c < NCH)
        def _():
            pltpu.make_async_copy(
                ctx_hbm.at[pl.ds(c * BC, BC), :], xbuf.at[slot], in_sem.at[slot]
            ).start()

        @pl.when(c >= NCH)
        def _():
            pltpu.make_async_copy(
                smp_hbm.at[pl.ds((c - NCH) * BC, BC), :], xbuf.at[slot], in_sem.at[slot]
            ).start()

    def wait_in(slot):
        pltpu.make_async_copy(
            ctx_hbm.at[pl.ds(0, BC), :], xbuf.at[slot], in_sem.at[slot]
        ).wait()

    def start_out(c, slot):
        pltpu.make_async_copy(
            obuf.at[slot], out_hbm.at[pl.ds(c * BC, BC), :], out_sem.at[slot]
        ).start()

    def wait_out(c, slot):
        pltpu.make_async_copy(
            obuf.at[slot], out_hbm.at[pl.ds(c * BC, BC), :], out_sem.at[slot]
        ).wait()

    for s in range(NBUF):
        start_in(s, s)

    def body(c, carry):
        slot = jax.lax.rem(c, NBUF)

        @pl.when(c >= NBUF)
        def _():
            wait_out(c - NBUF, slot)

        wait_in(slot)
        obuf[slot] = (
            jnp.dot(xbuf[slot], w_ref[...], preferred_element_type=jnp.float32)
            + b_ref[...]
        )
        start_out(c, slot)

        @pl.when(c + NBUF < NC)
        def _():
            start_in(c + NBUF, slot)

        return carry

    jax.lax.fori_loop(0, NC, body, 0)

    for k in range(NC - NBUF, NC):
        wait_out(k, k % NBUF)


def kernel(context, sample, W_proj, b_proj):
    b2d = b_proj.reshape(1, D)
    out = pl.pallas_call(
        _mm_kernel,
        in_specs=[
            pl.BlockSpec(memory_space=pl.ANY),
            pl.BlockSpec(memory_space=pl.ANY),
            pl.BlockSpec(memory_space=pltpu.VMEM),
            pl.BlockSpec(memory_space=pltpu.VMEM),
        ],
        out_specs=pl.BlockSpec(memory_space=pl.ANY),
        out_shape=jax.ShapeDtypeStruct((2 * N_HALF, D), jnp.float32),
        scratch_shapes=[
            pltpu.VMEM((NBUF, BC, D), jnp.float32),
            pltpu.VMEM((NBUF, BC, D), jnp.float32),
            pltpu.SemaphoreType.DMA((NBUF,)),
            pltpu.SemaphoreType.DMA((NBUF,)),
        ],
    )(context, sample, W_proj, b2d)
    return out


# speedup vs baseline: 1.1228x; 1.0601x over previous
import jax
import jax.numpy as jnp
from jax.experimental import pallas as pl
from jax.experimental.pallas import tpu as pltpu

N_HALF = 25000
D = 256
BC = 1000
NCH = N_HALF // BC
NC = 2 * NCH
NBUF = 8


def _mm_kernel(ctx_hbm, smp_hbm, w_ref, b_ref, out_hbm, xbuf, obuf, in_sem, out_sem):
    def start_in(c, slot):
        @pl.when(c < NCH)
        def _():
            pltpu.make_async_copy(
                ctx_hbm.at[pl.ds(c * BC, BC), :], xbuf.at[slot], in_sem.at[slot]
            ).start()

        @pl.when(c >= NCH)
        def _():
            pltpu.make_async_copy(
                smp_hbm.at[pl.ds((c - NCH) * BC, BC), :], xbuf.at[slot], in_sem.at[slot]
            ).start()

    def wait_in(slot):
        pltpu.make_async_copy(
            ctx_hbm.at[pl.ds(0, BC), :], xbuf.at[slot], in_sem.at[slot]
        ).wait()

    def start_out(c, slot):
        pltpu.make_async_copy(
            obuf.at[slot], out_hbm.at[pl.ds(c * BC, BC), :], out_sem.at[slot]
        ).start()

    def wait_out(c, slot):
        pltpu.make_async_copy(
            obuf.at[slot], out_hbm.at[pl.ds(c * BC, BC), :], out_sem.at[slot]
        ).wait()

    for s in range(NBUF):
        start_in(s, s)

    def body(c, carry):
        slot = jax.lax.rem(c, NBUF)

        @pl.when(c >= NBUF)
        def _():
            wait_out(c - NBUF, slot)

        wait_in(slot)
        obuf[slot] = (
            jnp.dot(xbuf[slot], w_ref[...], preferred_element_type=jnp.float32)
            + b_ref[...]
        )
        start_out(c, slot)

        @pl.when(c + NBUF < NC)
        def _():
            start_in(c + NBUF, slot)

        return carry

    jax.lax.fori_loop(0, NC, body, 0)

    for k in range(NC - NBUF, NC):
        wait_out(k, k % NBUF)


def kernel(context, sample, W_proj, b_proj):
    b2d = b_proj.reshape(1, D)
    out = pl.pallas_call(
        _mm_kernel,
        in_specs=[
            pl.BlockSpec(memory_space=pl.ANY),
            pl.BlockSpec(memory_space=pl.ANY),
            pl.BlockSpec(memory_space=pltpu.VMEM),
            pl.BlockSpec(memory_space=pltpu.VMEM),
        ],
        out_specs=pl.BlockSpec(memory_space=pl.ANY),
        out_shape=jax.ShapeDtypeStruct((2 * N_HALF, D), jnp.float32),
        scratch_shapes=[
            pltpu.VMEM((NBUF, BC, D), jnp.float32),
            pltpu.VMEM((NBUF, BC, D), jnp.float32),
            pltpu.SemaphoreType.DMA((NBUF,)),
            pltpu.SemaphoreType.DMA((NBUF,)),
        ],
    )(context, sample, W_proj, b2d)
    return out
